# baseline (speedup 1.0000x reference)
import functools

import jax
import jax.numpy as jnp
from jax import lax
from jax.experimental import pallas as pl
from jax.experimental.pallas import tpu as pltpu
from jax.experimental.pallas import tpu_sc as plsc

NC = 2
NS = 16
CH = 64
NB = 12
G = 64
F32 = jnp.float32


def _sc_mesh():
  return plsc.VectorSubcoreMesh(core_axis_name="c", subcore_axis_name="s")


def _row_split(N):
  r_sm = 8 * (N // (8 * NS))
  r_lg = N - r_sm * (NS - 1)
  return r_sm, r_lg


def _for_tile_rows(s, r_sm, r_lg, do):
  @pl.when(s < NS - 1)
  def _():
    do(pl.multiple_of(s * r_sm, 8), r_sm)

  @pl.when(s == NS - 1)
  def _():
    do((NS - 1) * r_sm, r_lg)


def _make_sc_edge_layer(N, H, E):
  NW = NC * NS
  epw = E // NW
  nchunk = epw // CH
  tail = epw - nchunk * CH
  nouter = nchunk // NB
  r_sm, r_lg = _row_split(N)

  @functools.partial(
      pl.kernel,
      out_type=jax.ShapeDtypeStruct((NC, N, H), F32),
      mesh=_sc_mesh(),
      compiler_params=pltpu.CompilerParams(use_tc_tiling_on_sc=False),
      scratch_types=[
          pltpu.VMEM((nchunk, CH), jnp.int32),
          pltpu.VMEM((nchunk, CH), jnp.int32),
          pltpu.VMEM((tail,), jnp.int32),
          pltpu.VMEM((tail,), jnp.int32),
          pltpu.VMEM((tail, H), F32),
          tuple(pltpu.VMEM((CH, H), F32) for _ in range(NB)),
          tuple(pltpu.SemaphoreType.DMA for _ in range(NB)),
          tuple(pltpu.SemaphoreType.DMA for _ in range(NB)),
          pltpu.VMEM_SHARED((N, H), F32),
      ],
  )
  def k(y_hbm, srcm, dstm, srct, dstt, zeros_hbm, out_hbm,
        srcv, dstv, srctv, dsttv, rowst, rows, gsem, ssem, acc):
    c = lax.axis_index("c")
    s = lax.axis_index("s")
    wid = c * NS + s
    i0 = pltpu.async_copy(srcm.at[wid], srcv, gsem[0])
    i1 = pltpu.async_copy(dstm.at[wid], dstv, gsem[1])
    i2 = pltpu.async_copy(srct.at[wid], srctv, gsem[2])
    i3 = pltpu.async_copy(dstt.at[wid], dsttv, gsem[3])
    _for_tile_rows(s, r_sm, r_lg, lambda o, n: pltpu.sync_copy(
        zeros_hbm.at[pl.ds(0, n)], acc.at[pl.ds(o, n)]))
    i0.wait()
    i1.wait()
    i2.wait()
    i3.wait()
    plsc.subcore_barrier()
    pltpu.sync_copy(y_hbm.at[srctv], rowst)
    pltpu.sync_copy(rowst, acc.at[dsttv], add=True)

    def g_start(j, b):
      pltpu.async_copy(y_hbm.at[srcv.at[j]], rows[b], gsem[b])

    def g_wait(j, b):
      pltpu.make_async_copy(y_hbm.at[srcv.at[j]], rows[b], gsem[b]).wait()

    def s_start(j, b):
      pltpu.async_copy(rows[b], acc.at[dstv.at[j]], ssem[b], add=True)

    def s_wait(j, b):
      pltpu.make_async_copy(rows[b], acc.at[dstv.at[j]], ssem[b]).wait()

    for b in range(NB - 1):
      g_start(b, b)

    def outer(i, carry):
      for b in range(NB):
        j = i * NB + b
        g_wait(j, b)
        s_start(j, b)
        b2 = (b + NB - 1) % NB
        jn = j + NB - 1
        if b == 0:

          @pl.when(i > 0)
          def _():
            s_wait(j - 1, b2)
            g_start(jn, b2)

          @pl.when(i == 0)
          def _():
            g_start(jn, b2)
        else:

          @pl.when(jn < nchunk)
          def _():
            s_wait(j - 1, b2)
            g_start(jn, b2)
      return carry

    lax.fori_loop(0, nouter, outer, 0)
    for b in range(NB):
      s_wait(nchunk - NB + b, b)
    plsc.subcore_barrier()
    _for_tile_rows(s, r_sm, r_lg, lambda o, n: pltpu.sync_copy(
        acc.at[pl.ds(o, n)], out_hbm.at[c].at[pl.ds(o, n)]))

  return k


def _make_sc_deg(N, E, W):
  NW = NC * NS
  epw = E // NW
  nchunk = epw // CH
  tail = epw - nchunk * CH
  r_sm, r_lg = _row_split(N)

  @functools.partial(
      pl.kernel,
      out_type=jax.ShapeDtypeStruct((NC, N, W), F32),
      mesh=_sc_mesh(),
      compiler_params=pltpu.CompilerParams(use_tc_tiling_on_sc=False),
      scratch_types=[
          pltpu.VMEM((nchunk, CH), jnp.int32),
          pltpu.VMEM((tail,), jnp.int32),
          pltpu.VMEM((CH, W), F32),
          pltpu.VMEM((tail, W), F32),
          tuple(pltpu.SemaphoreType.DMA for _ in range(NB)),
          pltpu.VMEM_SHARED((N, W), F32),
      ],
  )
  def k(dstm, dstt, ones_hbm, zeros_hbm, out_hbm, dstv, dsttv, ones_v, onest_v,
        ssem, acc):
    c = lax.axis_index("c")
    s = lax.axis_index("s")
    wid = c * NS + s
    i0 = pltpu.async_copy(dstm.at[wid], dstv, ssem[0])
    i1 = pltpu.async_copy(dstt.at[wid], dsttv, ssem[1])
    i2 = pltpu.async_copy(ones_hbm, ones_v, ssem[2])
    i3 = pltpu.async_copy(ones_hbm.at[pl.ds(0, tail)], onest_v, ssem[3])
    _for_tile_rows(s, r_sm, r_lg, lambda o, n: pltpu.sync_copy(
        zeros_hbm.at[pl.ds(0, n)], acc.at[pl.ds(o, n)]))
    i0.wait()
    i1.wait()
    i2.wait()
    i3.wait()
    plsc.subcore_barrier()
    pltpu.sync_copy(onest_v, acc.at[dsttv], add=True)

    def s_start(j, b):
      pltpu.async_copy(ones_v, acc.at[dstv.at[j]], ssem[b], add=True)

    def s_wait(j, b):
      pltpu.make_async_copy(ones_v, acc.at[dstv.at[j]], ssem[b]).wait()

    def outer(i, carry):
      for b in range(NB):
        j = i * NB + b

        @pl.when(i > 0)
        def _():
          s_wait(j - NB, b)

        s_start(j, b)
      return carry

    lax.fori_loop(0, nchunk // NB, outer, 0)
    for b in range(NB):
      s_wait(nchunk - NB + b, b)
    plsc.subcore_barrier()
    _for_tile_rows(s, r_sm, r_lg, lambda o, n: pltpu.sync_copy(
        acc.at[pl.ds(o, n)], out_hbm.at[c].at[pl.ds(o, n)]))

  return k


def _tc_first(x, W1, degp):
  N = x.shape[0]
  H = W1.shape[1]

  def body(x_ref, w_ref, degp_ref, dinv_ref, y_ref):
    deg = degp_ref[0, :, 0:1] + degp_ref[1, :, 0:1] + 1.0
    dinv = lax.rsqrt(deg)
    dinv_ref[...] = dinv
    xw = jnp.dot(x_ref[...], w_ref[...], preferred_element_type=F32)
    y_ref[...] = xw * dinv

  return pl.pallas_call(
      body,
      out_shape=(jax.ShapeDtypeStruct((N, 1), F32),
                 jax.ShapeDtypeStruct((N, H), F32)),
  )(x, W1, degp)


def _tc_mid(p, y, dinv, b, Wn):
  N, H = y.shape

  def body(p_ref, y_ref, dinv_ref, b_ref, w_ref, out_ref):
    acc = p_ref[0] + p_ref[1] + y_ref[...]
    x = acc * dinv_ref[...] + b_ref[...]
    out_ref[...] = jnp.dot(x, w_ref[...], preferred_element_type=F32) * dinv_ref[...]

  return pl.pallas_call(
      body, out_shape=jax.ShapeDtypeStruct((N, H), F32),
  )(p, y, dinv, b, Wn)


def _tc_last(p, y, dinv, b, seg, Wlin, blin):
  N, H = y.shape
  C = Wlin.shape[1]

  def body(p_ref, y_ref, dinv_ref, b_ref, seg_ref, wl_ref, bl_ref, out_ref):
    x3 = (p_ref[0] + p_ref[1] + y_ref[...]) * dinv_ref[...] + b_ref[...]
    gids = lax.broadcasted_iota(jnp.int32, (G, N), 0)
    M = (seg_ref[...] == gids).astype(F32)
    sums = jnp.dot(M, x3, preferred_element_type=F32)
    counts = jnp.sum(M, axis=1, keepdims=True)
    pooled = sums / jnp.maximum(counts, 1.0)
    out_ref[...] = jnp.dot(pooled, wl_ref[...], preferred_element_type=F32) + bl_ref[...]

  return pl.pallas_call(
      body, out_shape=jax.ShapeDtypeStruct((G, C), F32),
  )(p, y, dinv, b, seg, Wlin, blin)


def kernel(inputs, edge_index, batch_indexes, W1, b1, W2, b2, W3, b3, Wlin, blin):
  N, D = inputs.shape
  H = W1.shape[1]
  E = edge_index.shape[1]
  C = Wlin.shape[1]
  NW = NC * NS
  epw = E // NW
  nchunk = epw // CH
  e2 = edge_index.reshape(2, NW, epw)
  em = e2[:, :, :nchunk * CH].reshape(2, NW, nchunk, CH)
  et = e2[:, :, nchunk * CH:]
  srcm, dstm = em[0], em[1]
  srct, dstt = et[0], et[1]

  DW = 8
  _, r_lg = _row_split(N)
  zeros_h = jnp.zeros((r_lg, H), F32)
  zeros_d = jnp.zeros((r_lg, DW), F32)
  ones_d = jnp.ones((CH, DW), F32)

  sc_deg = _make_sc_deg(N, E, DW)
  sc_layer = _make_sc_edge_layer(N, H, E)

  degp = sc_deg(dstm, dstt, ones_d, zeros_d)
  dinv, y1 = _tc_first(inputs, W1, degp)
  p1 = sc_layer(y1, srcm, dstm, srct, dstt, zeros_h)
  y2 = _tc_mid(p1, y1, dinv, b1.reshape(1, H), W2)
  p2 = sc_layer(y2, srcm, dstm, srct, dstt, zeros_h)
  y3 = _tc_mid(p2, y2, dinv, b2.reshape(1, H), W3)
  p3 = sc_layer(y3, srcm, dstm, srct, dstt, zeros_h)
  seg = batch_indexes.reshape(1, N)
  return _tc_last(p3, y3, dinv, b3.reshape(1, H), seg, Wlin, blin.reshape(1, C))


# speedup vs baseline: 1.3149x; 1.3149x over previous
import functools

import jax
import jax.numpy as jnp
from jax import lax
from jax.experimental import pallas as pl
from jax.experimental.pallas import tpu as pltpu
from jax.experimental.pallas import tpu_sc as plsc

NC = 2
NS = 16
CH = 128
NB = 6
G = 64
F32 = jnp.float32


def _sc_mesh():
  return plsc.VectorSubcoreMesh(core_axis_name="c", subcore_axis_name="s")


def _row_split(N):
  r_sm = 8 * (N // (8 * NS))
  r_lg = N - r_sm * (NS - 1)
  return r_sm, r_lg


def _for_tile_rows(s, r_sm, r_lg, do):
  @pl.when(s < NS - 1)
  def _():
    do(pl.multiple_of(s * r_sm, 8), r_sm)

  @pl.when(s == NS - 1)
  def _():
    do((NS - 1) * r_sm, r_lg)


def _make_sc_edge_layer(N, H, E):
  NW = NC * NS
  epw = E // NW
  nchunk = epw // CH
  tail = epw - nchunk * CH
  nouter = nchunk // NB
  r_sm, r_lg = _row_split(N)

  @functools.partial(
      pl.kernel,
      out_type=jax.ShapeDtypeStruct((NC, N, H), F32),
      mesh=_sc_mesh(),
      compiler_params=pltpu.CompilerParams(use_tc_tiling_on_sc=False),
      scratch_types=[
          pltpu.VMEM((nchunk, CH), jnp.int32),
          pltpu.VMEM((nchunk, CH), jnp.int32),
          pltpu.VMEM((tail,), jnp.int32),
          pltpu.VMEM((tail,), jnp.int32),
          pltpu.VMEM((tail, H), F32),
          tuple(pltpu.VMEM((CH, H), F32) for _ in range(NB)),
          tuple(pltpu.SemaphoreType.DMA for _ in range(NB)),
          tuple(pltpu.SemaphoreType.DMA for _ in range(NB)),
          pltpu.VMEM_SHARED((N, H), F32),
      ],
  )
  def k(y_hbm, srcm, dstm, srct, dstt, zeros_hbm, out_hbm,
        srcv, dstv, srctv, dsttv, rowst, rows, gsem, ssem, acc):
    c = lax.axis_index("c")
    s = lax.axis_index("s")
    wid = c * NS + s
    i0 = pltpu.async_copy(srcm.at[wid], srcv, gsem[0])
    i1 = pltpu.async_copy(dstm.at[wid], dstv, gsem[1])
    i2 = pltpu.async_copy(srct.at[wid], srctv, gsem[2])
    i3 = pltpu.async_copy(dstt.at[wid], dsttv, gsem[3])
    _for_tile_rows(s, r_sm, r_lg, lambda o, n: pltpu.sync_copy(
        zeros_hbm.at[pl.ds(0, n)], acc.at[pl.ds(o, n)]))
    i0.wait()
    i1.wait()
    i2.wait()
    i3.wait()
    plsc.subcore_barrier()
    pltpu.sync_copy(y_hbm.at[srctv], rowst)
    pltpu.sync_copy(rowst, acc.at[dsttv], add=True)

    def g_start(j, b):
      pltpu.async_copy(y_hbm.at[srcv.at[j]], rows[b], gsem[b])

    def g_wait(j, b):
      pltpu.make_async_copy(y_hbm.at[srcv.at[j]], rows[b], gsem[b]).wait()

    def s_start(j, b):
      pltpu.async_copy(rows[b], acc.at[dstv.at[j]], ssem[b], add=True)

    def s_wait(j, b):
      pltpu.make_async_copy(rows[b], acc.at[dstv.at[j]], ssem[b]).wait()

    for b in range(NB - 1):
      g_start(b, b)

    def outer(i, carry):
      for b in range(NB):
        j = i * NB + b
        g_wait(j, b)
        s_start(j, b)
        b2 = (b + NB - 1) % NB
        jn = j + NB - 1
        if b == 0:

          @pl.when(i > 0)
          def _():
            s_wait(j - 1, b2)
            g_start(jn, b2)

          @pl.when(i == 0)
          def _():
            g_start(jn, b2)
        else:

          @pl.when(jn < nchunk)
          def _():
            s_wait(j - 1, b2)
            g_start(jn, b2)
      return carry

    lax.fori_loop(0, nouter, outer, 0)
    for b in range(NB):
      s_wait(nchunk - NB + b, b)
    plsc.subcore_barrier()
    _for_tile_rows(s, r_sm, r_lg, lambda o, n: pltpu.sync_copy(
        acc.at[pl.ds(o, n)], out_hbm.at[c].at[pl.ds(o, n)]))

  return k


def _make_sc_deg(N, E, W):
  NW = NC * NS
  epw = E // NW
  nchunk = epw // CH
  tail = epw - nchunk * CH
  r_sm, r_lg = _row_split(N)

  @functools.partial(
      pl.kernel,
      out_type=jax.ShapeDtypeStruct((NC, N, W), F32),
      mesh=_sc_mesh(),
      compiler_params=pltpu.CompilerParams(use_tc_tiling_on_sc=False),
      scratch_types=[
          pltpu.VMEM((nchunk, CH), jnp.int32),
          pltpu.VMEM((tail,), jnp.int32),
          pltpu.VMEM((CH, W), F32),
          pltpu.VMEM((tail, W), F32),
          tuple(pltpu.SemaphoreType.DMA for _ in range(NB)),
          pltpu.VMEM_SHARED((N, W), F32),
      ],
  )
  def k(dstm, dstt, ones_hbm, zeros_hbm, out_hbm, dstv, dsttv, ones_v, onest_v,
        ssem, acc):
    c = lax.axis_index("c")
    s = lax.axis_index("s")
    wid = c * NS + s
    i0 = pltpu.async_copy(dstm.at[wid], dstv, ssem[0])
    i1 = pltpu.async_copy(dstt.at[wid], dsttv, ssem[1])
    i2 = pltpu.async_copy(ones_hbm, ones_v, ssem[2])
    i3 = pltpu.async_copy(ones_hbm.at[pl.ds(0, tail)], onest_v, ssem[3])
    _for_tile_rows(s, r_sm, r_lg, lambda o, n: pltpu.sync_copy(
        zeros_hbm.at[pl.ds(0, n)], acc.at[pl.ds(o, n)]))
    i0.wait()
    i1.wait()
    i2.wait()
    i3.wait()
    plsc.subcore_barrier()
    pltpu.sync_copy(onest_v, acc.at[dsttv], add=True)

    def s_start(j, b):
      pltpu.async_copy(ones_v, acc.at[dstv.at[j]], ssem[b], add=True)

    def s_wait(j, b):
      pltpu.make_async_copy(ones_v, acc.at[dstv.at[j]], ssem[b]).wait()

    def outer(i, carry):
      for b in range(NB):
        j = i * NB + b

        @pl.when(i > 0)
        def _():
          s_wait(j - NB, b)

        s_start(j, b)
      return carry

    lax.fori_loop(0, nchunk // NB, outer, 0)
    for b in range(NB):
      s_wait(nchunk - NB + b, b)
    plsc.subcore_barrier()
    _for_tile_rows(s, r_sm, r_lg, lambda o, n: pltpu.sync_copy(
        acc.at[pl.ds(o, n)], out_hbm.at[c].at[pl.ds(o, n)]))

  return k


def _blockdiag2(w, z):
  return jnp.concatenate(
      [jnp.concatenate([w, z], axis=1), jnp.concatenate([z, w], axis=1)], axis=0)


def _tc_first(x2, W1, degp2, N, H, DW):
  NP = N // 2

  def body(x_ref, w_ref, degp_ref, dinv_ref, y_ref):
    degpair = degp_ref[0] + degp_ref[1]
    de = lax.rsqrt(degpair[:, 0:1] + 1.0)
    do = lax.rsqrt(degpair[:, DW:DW + 1] + 1.0)
    dinv2 = jnp.concatenate([jnp.broadcast_to(de, (NP, H)),
                             jnp.broadcast_to(do, (NP, H))], axis=1)
    dinv_ref[...] = dinv2
    z = jnp.zeros_like(w_ref[...])
    xw = jnp.dot(x_ref[...], _blockdiag2(w_ref[...], z),
                 preferred_element_type=F32)
    y_ref[...] = xw * dinv2

  return pl.pallas_call(
      body,
      out_shape=(jax.ShapeDtypeStruct((NP, 2 * H), F32),
                 jax.ShapeDtypeStruct((NP, 2 * H), F32)),
  )(x2, W1, degp2)


def _tc_mid(p2, y2, dinv2, b, Wn, N, H):
  def body(p_ref, y_ref, dinv_ref, b_ref, w_ref, out_ref):
    b2 = jnp.concatenate([b_ref[...], b_ref[...]], axis=1)
    x = (p_ref[0] + p_ref[1] + y_ref[...]) * dinv_ref[...] + b2
    z = jnp.zeros_like(w_ref[...])
    out_ref[...] = jnp.dot(x, _blockdiag2(w_ref[...], z),
                           preferred_element_type=F32) * dinv_ref[...]

  return pl.pallas_call(
      body, out_shape=jax.ShapeDtypeStruct((N // 2, 2 * H), F32),
  )(p2, y2, dinv2, b, Wn)


def _tc_last(p2, y2, dinv2, b, sege, sego, Wlin, blin, N, H):
  C = Wlin.shape[1]
  NP = N // 2

  def body(p_ref, y_ref, dinv_ref, b_ref, sege_ref, sego_ref, wl_ref, bl_ref,
           out_ref):
    b2 = jnp.concatenate([b_ref[...], b_ref[...]], axis=1)
    x3 = (p_ref[0] + p_ref[1] + y_ref[...]) * dinv_ref[...] + b2
    gids = lax.broadcasted_iota(jnp.int32, (G, NP), 0)
    Me = (sege_ref[...] == gids).astype(F32)
    Mo = (sego_ref[...] == gids).astype(F32)
    sums = (jnp.dot(Me, x3[:, 0:H], preferred_element_type=F32) +
            jnp.dot(Mo, x3[:, H:2 * H], preferred_element_type=F32))
    counts = jnp.sum(Me, axis=1, keepdims=True) + jnp.sum(Mo, axis=1, keepdims=True)
    pooled = sums / jnp.maximum(counts, 1.0)
    out_ref[...] = jnp.dot(pooled, wl_ref[...], preferred_element_type=F32) + bl_ref[...]

  return pl.pallas_call(
      body, out_shape=jax.ShapeDtypeStruct((G, C), F32),
  )(p2, y2, dinv2, b, sege, sego, Wlin, blin)


def kernel(inputs, edge_index, batch_indexes, W1, b1, W2, b2, W3, b3, Wlin, blin):
  N, D = inputs.shape
  H = W1.shape[1]
  E = edge_index.shape[1]
  C = Wlin.shape[1]
  NW = NC * NS
  epw = E // NW
  nchunk = epw // CH
  e2 = edge_index.reshape(2, NW, epw)
  em = e2[:, :, :nchunk * CH].reshape(2, NW, nchunk, CH)
  et = e2[:, :, nchunk * CH:]
  srcm, dstm = em[0], em[1]
  srct, dstt = et[0], et[1]

  DW = 8
  _, r_lg = _row_split(N)
  zeros_h = jnp.zeros((r_lg, H), F32)
  zeros_d = jnp.zeros((r_lg, DW), F32)
  ones_d = jnp.ones((CH, DW), F32)

  sc_deg = _make_sc_deg(N, E, DW)
  sc_layer = _make_sc_edge_layer(N, H, E)

  NP = N // 2

  def as_sc(y2d):
    return y2d.reshape(N, H)

  def as_tc(p):
    return p.reshape(NC, NP, 2 * H)

  x2 = inputs.reshape(NP, 2 * D)
  sege = batch_indexes[0::2].reshape(1, NP)
  sego = batch_indexes[1::2].reshape(1, NP)

  degp = sc_deg(dstm, dstt, ones_d, zeros_d)
  dinv2, y1 = _tc_first(x2, W1, degp.reshape(NC, NP, 2 * DW), N, H, DW)
  p1 = sc_layer(as_sc(y1), srcm, dstm, srct, dstt, zeros_h)
  y2 = _tc_mid(as_tc(p1), y1, dinv2, b1.reshape(1, H), W2, N, H)
  p2 = sc_layer(as_sc(y2), srcm, dstm, srct, dstt, zeros_h)
  y3 = _tc_mid(as_tc(p2), y2, dinv2, b2.reshape(1, H), W3, N, H)
  p3 = sc_layer(as_sc(y3), srcm, dstm, srct, dstt, zeros_h)
  return _tc_last(as_tc(p3), y3, dinv2, b3.reshape(1, H), sege, sego, Wlin,
                  blin.reshape(1, C), N, H)


# speedup vs baseline: 1.3551x; 1.0305x over previous
import functools

import jax
import jax.numpy as jnp
from jax import lax
from jax.experimental import pallas as pl
from jax.experimental.pallas import tpu as pltpu
from jax.experimental.pallas import tpu_sc as plsc

NC = 2
NS = 16
CH = 128
NB = 6
G = 64
F32 = jnp.float32


def _sc_mesh():
  return plsc.VectorSubcoreMesh(core_axis_name="c", subcore_axis_name="s")


def _row_split(N):
  r_sm = 8 * (N // (8 * NS))
  r_lg = N - r_sm * (NS - 1)
  return r_sm, r_lg


def _for_tile_rows(s, r_sm, r_lg, do):
  @pl.when(s < NS - 1)
  def _():
    do(pl.multiple_of(s * r_sm, 8), r_sm)

  @pl.when(s == NS - 1)
  def _():
    do((NS - 1) * r_sm, r_lg)


def _make_sc_edge_layer(N, H, E):
  NW = NC * NS
  tch = E // CH
  nchunk = tch // NW
  extra = tch - nchunk * NW
  nouter = nchunk // NB
  r_sm, r_lg = _row_split(N)

  @functools.partial(
      pl.kernel,
      out_type=jax.ShapeDtypeStruct((NC, N, H), F32),
      mesh=_sc_mesh(),
      compiler_params=pltpu.CompilerParams(use_tc_tiling_on_sc=False),
      scratch_types=[
          pltpu.VMEM((nchunk, 2, CH), jnp.int32),
          pltpu.VMEM((1, 2, CH), jnp.int32),
          tuple(pltpu.VMEM((CH, H), F32) for _ in range(NB)),
          tuple(pltpu.SemaphoreType.DMA for _ in range(NB)),
          tuple(pltpu.SemaphoreType.DMA for _ in range(NB)),
          pltpu.VMEM_SHARED((N, H), F32),
      ],
  )
  def k(y_hbm, ei3, zeros_hbm, out_hbm, eiv, extrav, rows, gsem, ssem, acc):
    c = lax.axis_index("c")
    s = lax.axis_index("s")
    wid = c * NS + s
    i0 = pltpu.async_copy(ei3.at[pl.ds(wid * nchunk, nchunk)], eiv, gsem[0])

    @pl.when(wid < extra)
    def _():
      pltpu.sync_copy(ei3.at[pl.ds(tch - extra + wid, 1)], extrav)

    _for_tile_rows(s, r_sm, r_lg, lambda o, n: pltpu.sync_copy(
        zeros_hbm.at[pl.ds(0, n)], acc.at[pl.ds(o, n)]))
    i0.wait()
    plsc.subcore_barrier()

    @pl.when(wid < extra)
    def _():
      pltpu.sync_copy(y_hbm.at[extrav.at[0, 0]], rows[NB - 1])
      pltpu.sync_copy(rows[NB - 1], acc.at[extrav.at[0, 1]], add=True)

    def g_start(j, b):
      pltpu.async_copy(y_hbm.at[eiv.at[j, 0]], rows[b], gsem[b])

    def g_wait(j, b):
      pltpu.make_async_copy(y_hbm.at[eiv.at[j, 0]], rows[b], gsem[b]).wait()

    def s_start(j, b):
      pltpu.async_copy(rows[b], acc.at[eiv.at[j, 1]], ssem[b], add=True)

    def s_wait(j, b):
      pltpu.make_async_copy(rows[b], acc.at[eiv.at[j, 1]], ssem[b]).wait()

    for b in range(NB - 1):
      g_start(b, b)

    def outer(i, carry):
      for b in range(NB):
        j = i * NB + b
        g_wait(j, b)
        s_start(j, b)
        b2 = (b + NB - 1) % NB
        jn = j + NB - 1
        if b == 0:

          @pl.when(i > 0)
          def _():
            s_wait(j - 1, b2)
            g_start(jn, b2)

          @pl.when(i == 0)
          def _():
            g_start(jn, b2)
        else:

          @pl.when(jn < nchunk)
          def _():
            s_wait(j - 1, b2)
            g_start(jn, b2)
      return carry

    lax.fori_loop(0, nouter, outer, 0)
    for b in range(NB):
      s_wait(nchunk - NB + b, b)
    plsc.subcore_barrier()
    _for_tile_rows(s, r_sm, r_lg, lambda o, n: pltpu.sync_copy(
        acc.at[pl.ds(o, n)], out_hbm.at[c].at[pl.ds(o, n)]))

  return k


def _make_sc_deg(N, E, W):
  NW = NC * NS
  tch = E // CH
  nchunk = tch // NW
  extra = tch - nchunk * NW
  r_sm, r_lg = _row_split(N)

  @functools.partial(
      pl.kernel,
      out_type=jax.ShapeDtypeStruct((NC, N, W), F32),
      mesh=_sc_mesh(),
      compiler_params=pltpu.CompilerParams(use_tc_tiling_on_sc=False),
      scratch_types=[
          pltpu.VMEM((nchunk, 2, CH), jnp.int32),
          pltpu.VMEM((1, 2, CH), jnp.int32),
          pltpu.VMEM((CH, W), F32),
          tuple(pltpu.SemaphoreType.DMA for _ in range(NB)),
          pltpu.VMEM_SHARED((N, W), F32),
      ],
  )
  def k(ei3, ones_hbm, zeros_hbm, out_hbm, eiv, extrav, ones_v, ssem, acc):
    c = lax.axis_index("c")
    s = lax.axis_index("s")
    wid = c * NS + s
    i0 = pltpu.async_copy(ei3.at[pl.ds(wid * nchunk, nchunk)], eiv, ssem[0])
    i2 = pltpu.async_copy(ones_hbm, ones_v, ssem[2])

    @pl.when(wid < extra)
    def _():
      pltpu.sync_copy(ei3.at[pl.ds(tch - extra + wid, 1)], extrav)

    _for_tile_rows(s, r_sm, r_lg, lambda o, n: pltpu.sync_copy(
        zeros_hbm.at[pl.ds(0, n)], acc.at[pl.ds(o, n)]))
    i0.wait()
    i2.wait()
    plsc.subcore_barrier()

    @pl.when(wid < extra)
    def _():
      pltpu.sync_copy(ones_v, acc.at[extrav.at[0, 1]], add=True)

    def s_start(j, b):
      pltpu.async_copy(ones_v, acc.at[eiv.at[j, 1]], ssem[b], add=True)

    def s_wait(j, b):
      pltpu.make_async_copy(ones_v, acc.at[eiv.at[j, 1]], ssem[b]).wait()

    def outer(i, carry):
      for b in range(NB):
        j = i * NB + b

        @pl.when(i > 0)
        def _():
          s_wait(j - NB, b)

        s_start(j, b)
      return carry

    lax.fori_loop(0, nchunk // NB, outer, 0)
    for b in range(NB):
      s_wait(nchunk - NB + b, b)
    plsc.subcore_barrier()
    _for_tile_rows(s, r_sm, r_lg, lambda o, n: pltpu.sync_copy(
        acc.at[pl.ds(o, n)], out_hbm.at[c].at[pl.ds(o, n)]))

  return k


def _blockdiag2(w, z):
  return jnp.concatenate(
      [jnp.concatenate([w, z], axis=1), jnp.concatenate([z, w], axis=1)], axis=0)


def _tc_first(x2, W1, degp2, N, H, DW):
  NP = N // 2

  def body(x_ref, w_ref, degp_ref, dinv_ref, y_ref):
    degpair = degp_ref[0] + degp_ref[1]
    de = lax.rsqrt(degpair[:, 0:1] + 1.0)
    do = lax.rsqrt(degpair[:, DW:DW + 1] + 1.0)
    dinv2 = jnp.concatenate([jnp.broadcast_to(de, (NP, H)),
                             jnp.broadcast_to(do, (NP, H))], axis=1)
    dinv_ref[...] = dinv2
    z = jnp.zeros_like(w_ref[...])
    xw = jnp.dot(x_ref[...], _blockdiag2(w_ref[...], z),
                 preferred_element_type=F32)
    y_ref[...] = xw * dinv2

  return pl.pallas_call(
      body,
      out_shape=(jax.ShapeDtypeStruct((NP, 2 * H), F32),
                 jax.ShapeDtypeStruct((NP, 2 * H), F32)),
  )(x2, W1, degp2)


def _tc_mid(p2, y2, dinv2, b, Wn, N, H):
  def body(p_ref, y_ref, dinv_ref, b_ref, w_ref, out_ref):
    b2 = jnp.concatenate([b_ref[...], b_ref[...]], axis=1)
    x = (p_ref[0] + p_ref[1] + y_ref[...]) * dinv_ref[...] + b2
    z = jnp.zeros_like(w_ref[...])
    out_ref[...] = jnp.dot(x, _blockdiag2(w_ref[...], z),
                           preferred_element_type=F32) * dinv_ref[...]

  return pl.pallas_call(
      body, out_shape=jax.ShapeDtypeStruct((N // 2, 2 * H), F32),
  )(p2, y2, dinv2, b, Wn)


def _tc_last(p2, y2, dinv2, b, sege, sego, Wlin, blin, N, H):
  C = Wlin.shape[1]
  NP = N // 2

  def body(p_ref, y_ref, dinv_ref, b_ref, sege_ref, sego_ref, wl_ref, bl_ref,
           out_ref):
    b2 = jnp.concatenate([b_ref[...], b_ref[...]], axis=1)
    x3 = (p_ref[0] + p_ref[1] + y_ref[...]) * dinv_ref[...] + b2
    gids = lax.broadcasted_iota(jnp.int32, (G, NP), 0)
    Me = (sege_ref[...] == gids).astype(F32)
    Mo = (sego_ref[...] == gids).astype(F32)
    sums = (jnp.dot(Me, x3[:, 0:H], preferred_element_type=F32) +
            jnp.dot(Mo, x3[:, H:2 * H], preferred_element_type=F32))
    counts = jnp.sum(Me, axis=1, keepdims=True) + jnp.sum(Mo, axis=1, keepdims=True)
    pooled = sums / jnp.maximum(counts, 1.0)
    out_ref[...] = jnp.dot(pooled, wl_ref[...], preferred_element_type=F32) + bl_ref[...]

  return pl.pallas_call(
      body, out_shape=jax.ShapeDtypeStruct((G, C), F32),
  )(p2, y2, dinv2, b, sege, sego, Wlin, blin)


def kernel(inputs, edge_index, batch_indexes, W1, b1, W2, b2, W3, b3, Wlin, blin):
  N, D = inputs.shape
  H = W1.shape[1]
  E = edge_index.shape[1]
  C = Wlin.shape[1]
  ei3 = jnp.swapaxes(edge_index.reshape(2, E // CH, CH), 0, 1)

  DW = 8
  _, r_lg = _row_split(N)
  zeros_h = jnp.zeros((r_lg, H), F32)
  zeros_d = jnp.zeros((r_lg, DW), F32)
  ones_d = jnp.ones((CH, DW), F32)

  sc_deg = _make_sc_deg(N, E, DW)
  sc_layer = _make_sc_edge_layer(N, H, E)

  NP = N // 2

  def as_sc(y2d):
    return y2d.reshape(N, H)

  def as_tc(p):
    return p.reshape(NC, NP, 2 * H)

  x2 = inputs.reshape(NP, 2 * D)
  sege = batch_indexes[0::2].reshape(1, NP)
  sego = batch_indexes[1::2].reshape(1, NP)

  degp = sc_deg(ei3, ones_d, zeros_d)
  dinv2, y1 = _tc_first(x2, W1, degp.reshape(NC, NP, 2 * DW), N, H, DW)
  p1 = sc_layer(as_sc(y1), ei3, zeros_h)
  y2 = _tc_mid(as_tc(p1), y1, dinv2, b1.reshape(1, H), W2, N, H)
  p2 = sc_layer(as_sc(y2), ei3, zeros_h)
  y3 = _tc_mid(as_tc(p2), y2, dinv2, b2.reshape(1, H), W3, N, H)
  p3 = sc_layer(as_sc(y3), ei3, zeros_h)
  return _tc_last(as_tc(p3), y3, dinv2, b3.reshape(1, H), sege, sego, Wlin,
                  blin.reshape(1, C), N, H)


# speedup vs baseline: 1.3560x; 1.0007x over previous
import functools

import jax
import jax.numpy as jnp
from jax import lax
from jax.experimental import pallas as pl
from jax.experimental.pallas import tpu as pltpu
from jax.experimental.pallas import tpu_sc as plsc

NC = 2
NS = 16
CH = 128
NB = 6
G = 64
F32 = jnp.float32


def _sc_mesh():
  return plsc.VectorSubcoreMesh(core_axis_name="c", subcore_axis_name="s")


def _row_split(N):
  r_sm = 8 * (N // (8 * NS))
  r_lg = N - r_sm * (NS - 1)
  return r_sm, r_lg


def _for_tile_rows(s, r_sm, r_lg, do):
  @pl.when(s < NS - 1)
  def _():
    do(pl.multiple_of(s * r_sm, 8), r_sm)

  @pl.when(s == NS - 1)
  def _():
    do((NS - 1) * r_sm, r_lg)


def _make_sc_edge_layer(N, H, E):
  NW = NC * NS
  tch = E // CH
  nchunk = tch // NW
  extra = tch - nchunk * NW
  nouter = nchunk // NB
  r_sm, r_lg = _row_split(N)

  @functools.partial(
      pl.kernel,
      out_type=jax.ShapeDtypeStruct((NC, N, H), F32),
      mesh=_sc_mesh(),
      compiler_params=pltpu.CompilerParams(use_tc_tiling_on_sc=False),
      scratch_types=[
          pltpu.VMEM((nchunk, 2, CH), jnp.int32),
          pltpu.VMEM((1, 2, CH), jnp.int32),
          tuple(pltpu.VMEM((CH, H), F32) for _ in range(NB)),
          tuple(pltpu.SemaphoreType.DMA for _ in range(NB)),
          tuple(pltpu.SemaphoreType.DMA for _ in range(NB)),
          pltpu.VMEM_SHARED((N, H), F32),
      ],
  )
  def k(y_hbm, ei3, zeros_hbm, out_hbm, eiv, extrav, rows, gsem, ssem, acc):
    c = lax.axis_index("c")
    s = lax.axis_index("s")
    wid = c * NS + s
    i0 = pltpu.async_copy(ei3.at[pl.ds(wid * nchunk, nchunk)], eiv, gsem[0])

    @pl.when(wid < extra)
    def _():
      pltpu.sync_copy(ei3.at[pl.ds(tch - extra + wid, 1)], extrav)

    _for_tile_rows(s, r_sm, r_lg, lambda o, n: pltpu.sync_copy(
        zeros_hbm.at[pl.ds(0, n)], acc.at[pl.ds(o, n)]))
    i0.wait()
    plsc.subcore_barrier()

    @pl.when(wid < extra)
    def _():
      pltpu.sync_copy(y_hbm.at[extrav.at[0, 0]], rows[NB - 1])
      pltpu.sync_copy(rows[NB - 1], acc.at[extrav.at[0, 1]], add=True)

    def g_start(j, b):
      pltpu.async_copy(y_hbm.at[eiv.at[j, 0]], rows[b], gsem[b])

    def g_wait(j, b):
      pltpu.make_async_copy(y_hbm.at[eiv.at[j, 0]], rows[b], gsem[b]).wait()

    def s_start(j, b):
      pltpu.async_copy(rows[b], acc.at[eiv.at[j, 1]], ssem[b], add=True)

    def s_wait(j, b):
      pltpu.make_async_copy(rows[b], acc.at[eiv.at[j, 1]], ssem[b]).wait()

    AH = NB - 2
    for b in range(AH):
      g_start(b, b)

    def outer(i, carry):
      for b in range(NB):
        j = i * NB + b
        g_wait(j, b)
        s_start(j, b)
        jn = j + AH
        b2 = (b + AH) % NB
        jp = jn - NB
        if b < NB - AH:

          @pl.when(i > 0)
          def _():
            s_wait(jp, b2)
            g_start(jn, b2)

          @pl.when(i == 0)
          def _():
            g_start(jn, b2)
        else:

          @pl.when(jn < nchunk)
          def _():
            s_wait(jp, b2)
            g_start(jn, b2)
      return carry

    lax.fori_loop(0, nouter, outer, 0)
    for j in range(nchunk - NB, nchunk):
      s_wait(j, j % NB)
    plsc.subcore_barrier()
    _for_tile_rows(s, r_sm, r_lg, lambda o, n: pltpu.sync_copy(
        acc.at[pl.ds(o, n)], out_hbm.at[c].at[pl.ds(o, n)]))

  return k


def _make_sc_deg(N, E, W):
  NW = NC * NS
  tch = E // CH
  nchunk = tch // NW
  extra = tch - nchunk * NW
  r_sm, r_lg = _row_split(N)

  @functools.partial(
      pl.kernel,
      out_type=jax.ShapeDtypeStruct((NC, N, W), F32),
      mesh=_sc_mesh(),
      compiler_params=pltpu.CompilerParams(use_tc_tiling_on_sc=False),
      scratch_types=[
          pltpu.VMEM((nchunk, 2, CH), jnp.int32),
          pltpu.VMEM((1, 2, CH), jnp.int32),
          pltpu.VMEM((CH, W), F32),
          tuple(pltpu.SemaphoreType.DMA for _ in range(NB)),
          pltpu.VMEM_SHARED((N, W), F32),
      ],
  )
  def k(ei3, ones_hbm, zeros_hbm, out_hbm, eiv, extrav, ones_v, ssem, acc):
    c = lax.axis_index("c")
    s = lax.axis_index("s")
    wid = c * NS + s
    i0 = pltpu.async_copy(ei3.at[pl.ds(wid * nchunk, nchunk)], eiv, ssem[0])
    i2 = pltpu.async_copy(ones_hbm, ones_v, ssem[2])

    @pl.when(wid < extra)
    def _():
      pltpu.sync_copy(ei3.at[pl.ds(tch - extra + wid, 1)], extrav)

    _for_tile_rows(s, r_sm, r_lg, lambda o, n: pltpu.sync_copy(
        zeros_hbm.at[pl.ds(0, n)], acc.at[pl.ds(o, n)]))
    i0.wait()
    i2.wait()
    plsc.subcore_barrier()

    @pl.when(wid < extra)
    def _():
      pltpu.sync_copy(ones_v, acc.at[extrav.at[0, 1]], add=True)

    def s_start(j, b):
      pltpu.async_copy(ones_v, acc.at[eiv.at[j, 1]], ssem[b], add=True)

    def s_wait(j, b):
      pltpu.make_async_copy(ones_v, acc.at[eiv.at[j, 1]], ssem[b]).wait()

    def outer(i, carry):
      for b in range(NB):
        j = i * NB + b

        @pl.when(i > 0)
        def _():
          s_wait(j - NB, b)

        s_start(j, b)
      return carry

    lax.fori_loop(0, nchunk // NB, outer, 0)
    for b in range(NB):
      s_wait(nchunk - NB + b, b)
    plsc.subcore_barrier()
    _for_tile_rows(s, r_sm, r_lg, lambda o, n: pltpu.sync_copy(
        acc.at[pl.ds(o, n)], out_hbm.at[c].at[pl.ds(o, n)]))

  return k


def _blockdiag2(w, z):
  return jnp.concatenate(
      [jnp.concatenate([w, z], axis=1), jnp.concatenate([z, w], axis=1)], axis=0)


def _tc_first(x2, W1, degp2, N, H, DW):
  NP = N // 2

  def body(x_ref, w_ref, degp_ref, dinv_ref, y_ref):
    degpair = degp_ref[0] + degp_ref[1]
    de = lax.rsqrt(degpair[:, 0:1] + 1.0)
    do = lax.rsqrt(degpair[:, DW:DW + 1] + 1.0)
    dinv2 = jnp.concatenate([jnp.broadcast_to(de, (NP, H)),
                             jnp.broadcast_to(do, (NP, H))], axis=1)
    dinv_ref[...] = dinv2
    z = jnp.zeros_like(w_ref[...])
    xw = jnp.dot(x_ref[...], _blockdiag2(w_ref[...], z),
                 preferred_element_type=F32)
    y_ref[...] = xw * dinv2

  return pl.pallas_call(
      body,
      out_shape=(jax.ShapeDtypeStruct((NP, 2 * H), F32),
                 jax.ShapeDtypeStruct((NP, 2 * H), F32)),
  )(x2, W1, degp2)


def _tc_mid(p2, y2, dinv2, b, Wn, N, H):
  def body(p_ref, y_ref, dinv_ref, b_ref, w_ref, out_ref):
    b2 = jnp.concatenate([b_ref[...], b_ref[...]], axis=1)
    x = (p_ref[0] + p_ref[1] + y_ref[...]) * dinv_ref[...] + b2
    z = jnp.zeros_like(w_ref[...])
    out_ref[...] = jnp.dot(x, _blockdiag2(w_ref[...], z),
                           preferred_element_type=F32) * dinv_ref[...]

  return pl.pallas_call(
      body, out_shape=jax.ShapeDtypeStruct((N // 2, 2 * H), F32),
  )(p2, y2, dinv2, b, Wn)


def _tc_last(p2, y2, dinv2, b, sege, sego, Wlin, blin, N, H):
  C = Wlin.shape[1]
  NP = N // 2

  def body(p_ref, y_ref, dinv_ref, b_ref, sege_ref, sego_ref, wl_ref, bl_ref,
           out_ref):
    b2 = jnp.concatenate([b_ref[...], b_ref[...]], axis=1)
    x3 = (p_ref[0] + p_ref[1] + y_ref[...]) * dinv_ref[...] + b2
    gids = lax.broadcasted_iota(jnp.int32, (G, NP), 0)
    Me = (sege_ref[...] == gids).astype(F32)
    Mo = (sego_ref[...] == gids).astype(F32)
    sums = (jnp.dot(Me, x3[:, 0:H], preferred_element_type=F32) +
            jnp.dot(Mo, x3[:, H:2 * H], preferred_element_type=F32))
    counts = jnp.sum(Me, axis=1, keepdims=True) + jnp.sum(Mo, axis=1, keepdims=True)
    pooled = sums / jnp.maximum(counts, 1.0)
    out_ref[...] = jnp.dot(pooled, wl_ref[...], preferred_element_type=F32) + bl_ref[...]

  return pl.pallas_call(
      body, out_shape=jax.ShapeDtypeStruct((G, C), F32),
  )(p2, y2, dinv2, b, sege, sego, Wlin, blin)


def kernel(inputs, edge_index, batch_indexes, W1, b1, W2, b2, W3, b3, Wlin, blin):
  N, D = inputs.shape
  H = W1.shape[1]
  E = edge_index.shape[1]
  C = Wlin.shape[1]
  ei3 = jnp.swapaxes(edge_index.reshape(2, E // CH, CH), 0, 1)

  DW = 8
  _, r_lg = _row_split(N)
  zeros_h = jnp.zeros((r_lg, H), F32)
  zeros_d = jnp.zeros((r_lg, DW), F32)
  ones_d = jnp.ones((CH, DW), F32)

  sc_deg = _make_sc_deg(N, E, DW)
  sc_layer = _make_sc_edge_layer(N, H, E)

  NP = N // 2

  def as_sc(y2d):
    return y2d.reshape(N, H)

  def as_tc(p):
    return p.reshape(NC, NP, 2 * H)

  x2 = inputs.reshape(NP, 2 * D)
  sege = batch_indexes[0::2].reshape(1, NP)
  sego = batch_indexes[1::2].reshape(1, NP)

  degp = sc_deg(ei3, ones_d, zeros_d)
  dinv2, y1 = _tc_first(x2, W1, degp.reshape(NC, NP, 2 * DW), N, H, DW)
  p1 = sc_layer(as_sc(y1), ei3, zeros_h)
  y2 = _tc_mid(as_tc(p1), y1, dinv2, b1.reshape(1, H), W2, N, H)
  p2 = sc_layer(as_sc(y2), ei3, zeros_h)
  y3 = _tc_mid(as_tc(p2), y2, dinv2, b2.reshape(1, H), W3, N, H)
  p3 = sc_layer(as_sc(y3), ei3, zeros_h)
  return _tc_last(as_tc(p3), y3, dinv2, b3.reshape(1, H), sege, sego, Wlin,
                  blin.reshape(1, C), N, H)
